# baseline (speedup 1.0000x reference)
import jax
import jax.numpy as jnp
from jax.experimental import pallas as pl
from jax.experimental.pallas import tpu as pltpu

_N = 32
_K = 12
_W = 2 ** _K


def _decode_kernel(noisy_hbm, g_ref, sig_ref, out_hbm,
                   x_vmem, o_vmem, sem_in, sem_out):
    cp_in = pltpu.make_async_copy(noisy_hbm, x_vmem, sem_in)
    cp_in.start()

    gf = g_ref[...].astype(jnp.float32)
    w_ids = jax.lax.broadcasted_iota(jnp.int32, (_K, _W), 1)
    j_ids = jax.lax.broadcasted_iota(jnp.int32, (_K, _W), 0)
    bits_t = ((w_ids >> j_ids) & 1).astype(jnp.float32)
    c_t = jax.lax.dot_general(
        gf, bits_t, (((0,), (0,)), ((), ())),
        preferred_element_type=jnp.float32)
    c_t = c_t - 2.0 * jnp.floor(c_t * 0.5)
    s_bf = (1.0 - 2.0 * c_t).astype(jnp.bfloat16)
    sc = jnp.concatenate([s_bf, s_bf, s_bf], axis=0)

    cp_in.wait()
    x = x_vmem[...] * (-4.0 / sig_ref[0, 0])

    x1 = x.astype(jnp.bfloat16)
    r1 = x - x1.astype(jnp.float32)
    x2 = r1.astype(jnp.bfloat16)
    x3 = (r1 - x2.astype(jnp.float32)).astype(jnp.bfloat16)
    xc = jnp.concatenate([x1, x2, x3], axis=1)
    scores = jnp.dot(xc, sc, preferred_element_type=jnp.float32)

    idx = jnp.argmax(scores, axis=1).astype(jnp.int32)[:, None]

    jbit = jax.lax.broadcasted_iota(jnp.int32, (scores.shape[0], _K), 1)
    o_vmem[...] = ((idx >> jbit) & 1).astype(jnp.float32)

    cp_out = pltpu.make_async_copy(o_vmem, out_hbm, sem_out)
    cp_out.start()
    cp_out.wait()


def kernel(noisy_symbols, G, sigma2):
    noisy = noisy_symbols
    b = noisy.shape[0]
    sig = jnp.reshape(sigma2.astype(jnp.float32), (1, 1))
    return pl.pallas_call(
        _decode_kernel,
        in_specs=[
            pl.BlockSpec(memory_space=pl.ANY),
            pl.BlockSpec(memory_space=pltpu.VMEM),
            pl.BlockSpec(memory_space=pltpu.VMEM),
        ],
        out_specs=pl.BlockSpec(memory_space=pl.ANY),
        out_shape=jax.ShapeDtypeStruct((b, _K), jnp.float32),
        scratch_shapes=[
            pltpu.VMEM((b, _N), jnp.float32),
            pltpu.VMEM((b, _K), jnp.float32),
            pltpu.SemaphoreType.DMA,
            pltpu.SemaphoreType.DMA,
        ],
    )(noisy, G, sig)
